# Initial kernel scaffold; baseline (speedup 1.0000x reference)
import jax
import jax.numpy as jnp
from jax.experimental import pallas as pl


def kernel(z, embedding):
    raise NotImplementedError("write your pallas kernel here")


# speedup vs baseline: 1.0591x; 1.0591x over previous
import functools

import jax
import jax.numpy as jnp
from jax.experimental import pallas as pl
from jax.experimental.pallas import tpu as pltpu

_BETA = 0.25


def _vq_body(z_ref, e_ref, out_ref, idx_ref, loss_ref):
    zb = z_ref[0]
    e = e_ref[...]
    en = e / jnp.maximum(
        jnp.sqrt(jnp.sum(e * e, axis=1, keepdims=True)), 1e-12)
    e2 = jnp.sum(en * en, axis=1)

    nz = jnp.sqrt(jnp.sum(zb * zb, axis=0, keepdims=True))
    zn = zb / jnp.maximum(nz, 1e-12)
    zr = zn.T
    z2 = jnp.sum(zr * zr, axis=1, keepdims=True)

    scores = jnp.dot(zr, en.T, preferred_element_type=jnp.float32)
    dist = z2 + e2[None, :] - 2.0 * scores

    m = jnp.min(dist, axis=1, keepdims=True)
    iota = jax.lax.broadcasted_iota(jnp.int32, dist.shape, 1)
    idx = jnp.min(jnp.where(dist == m, iota, jnp.int32(2**30)), axis=1)
    idx_ref[0] = idx.reshape(1, -1)

    onehot = (iota == idx[:, None]).astype(jnp.float32)
    zq = jnp.dot(onehot, en, preferred_element_type=jnp.float32,
                 precision=jax.lax.Precision.HIGHEST)

    out_ref[0] = zn + (zq.T - zn)
    loss_ref[0] = jnp.full((1, 128), jnp.sum((zq - zr) ** 2), jnp.float32)


@jax.jit
def kernel(z, embedding):
    b, d, h, w = z.shape
    k = embedding.shape[0]
    p = h * w
    z3 = z.reshape(b, d, p)

    out, idx, losssum = pl.pallas_call(
        _vq_body,
        grid=(b,),
        in_specs=[
            pl.BlockSpec((1, d, p), lambda i: (i, 0, 0)),
            pl.BlockSpec((k, d), lambda i: (0, 0)),
        ],
        out_specs=[
            pl.BlockSpec((1, d, p), lambda i: (i, 0, 0)),
            pl.BlockSpec((1, 1, p), lambda i: (i, 0, 0)),
            pl.BlockSpec((1, 1, 128), lambda i: (i, 0, 0)),
        ],
        out_shape=[
            jax.ShapeDtypeStruct((b, d, p), jnp.float32),
            jax.ShapeDtypeStruct((b, 1, p), jnp.int32),
            jax.ShapeDtypeStruct((b, 1, 128), jnp.float32),
        ],
        compiler_params=pltpu.CompilerParams(
            dimension_semantics=("arbitrary",)),
    )(z3, embedding)

    mean_sq = jnp.sum(losssum[:, 0, 0]) / jnp.float32(b * p * d)
    loss = mean_sq + _BETA * mean_sq
    return (out.reshape(b, d, h, w), idx.reshape(b * p), loss)


# speedup vs baseline: 1.0739x; 1.0140x over previous
import functools

import jax
import jax.numpy as jnp
from jax import lax
from jax.experimental import pallas as pl
from jax.experimental.pallas import tpu as pltpu
from jax.experimental.pallas import tpu_sc as plsc

_BETA = 0.25


def _dist_body(z_ref, e_ref, idx_ref, ent_ref, loss_ref):
    zb = z_ref[0]
    e = e_ref[...]
    en = e / jnp.maximum(
        jnp.sqrt(jnp.sum(e * e, axis=1, keepdims=True)), 1e-12)
    e2 = jnp.sum(en * en, axis=1)

    nz = jnp.sqrt(jnp.sum(zb * zb, axis=0, keepdims=True))
    zn = zb / jnp.maximum(nz, 1e-12)
    zr = zn.T
    z2 = jnp.sum(zr * zr, axis=1, keepdims=True)

    scores = jnp.dot(zr, en.T, preferred_element_type=jnp.float32)
    dist = z2 + e2[None, :] - 2.0 * scores

    m = jnp.min(dist, axis=1, keepdims=True)
    iota = jax.lax.broadcasted_iota(jnp.int32, dist.shape, 1)
    idx = jnp.min(jnp.where(dist == m, iota, jnp.int32(2**30)), axis=1)
    idx_ref[0] = idx.reshape(1, -1)
    loss_ref[0] = jnp.full((1, 128), jnp.sum(m), jnp.float32)

    @pl.when(pl.program_id(0) == 0)
    def _():
        ent_ref[...] = en


def _gather_body(en_hbm, idx_hbm, rows_hbm, idx_v, rows_v, sem):
    wid = lax.axis_index("s") * 2 + lax.axis_index("c")
    base = wid * 256
    pltpu.sync_copy(idx_hbm.at[pl.ds(base, 256)], idx_v)
    pltpu.async_copy(en_hbm.at[idx_v], rows_v, sem).wait()
    pltpu.sync_copy(rows_v, rows_hbm.at[pl.ds(base, 256)])


def _transpose_body(rows_ref, out_ref):
    out_ref[0] = rows_ref[0].T


@jax.jit
def kernel(z, embedding):
    b, d, h, w = z.shape
    k = embedding.shape[0]
    p = h * w
    z3 = z.reshape(b, d, p)

    idx, ent, losssum = pl.pallas_call(
        _dist_body,
        grid=(b,),
        in_specs=[
            pl.BlockSpec((1, d, p), lambda i: (i, 0, 0)),
            pl.BlockSpec((k, d), lambda i: (0, 0)),
        ],
        out_specs=[
            pl.BlockSpec((1, 1, p), lambda i: (i, 0, 0)),
            pl.BlockSpec((k, d), lambda i: (0, 0)),
            pl.BlockSpec((1, 1, 128), lambda i: (i, 0, 0)),
        ],
        out_shape=[
            jax.ShapeDtypeStruct((b, 1, p), jnp.int32),
            jax.ShapeDtypeStruct((k, d), jnp.float32),
            jax.ShapeDtypeStruct((b, 1, 128), jnp.float32),
        ],
        compiler_params=pltpu.CompilerParams(
            dimension_semantics=("arbitrary",)),
    )(z3, embedding)

    idx_flat = idx.reshape(b * p)
    gather = functools.partial(
        pl.kernel,
        out_type=jax.ShapeDtypeStruct((b * p, d), jnp.float32),
        mesh=plsc.VectorSubcoreMesh(core_axis_name="c", subcore_axis_name="s"),
        scratch_types=[
            pltpu.VMEM((256,), jnp.int32),
            pltpu.VMEM((256, d), jnp.float32),
            pltpu.SemaphoreType.DMA,
        ],
    )(_gather_body)
    rows = gather(ent, idx_flat)

    out = pl.pallas_call(
        _transpose_body,
        grid=(b,),
        in_specs=[pl.BlockSpec((1, p, d), lambda i: (i, 0, 0))],
        out_specs=pl.BlockSpec((1, d, p), lambda i: (i, 0, 0)),
        out_shape=jax.ShapeDtypeStruct((b, d, p), jnp.float32),
        compiler_params=pltpu.CompilerParams(
            dimension_semantics=("arbitrary",)),
    )(rows.reshape(b, p, d))

    mean_sq = jnp.sum(losssum[:, 0, 0]) / jnp.float32(b * p * d)
    loss = mean_sq + _BETA * mean_sq
    return (out.reshape(b, d, h, w), idx.reshape(b * p), loss)


# speedup vs baseline: 1.1376x; 1.0593x over previous
import functools

import jax
import jax.numpy as jnp
from jax import lax
from jax.experimental import pallas as pl
from jax.experimental.pallas import tpu as pltpu
from jax.experimental.pallas import tpu_sc as plsc

_BETA = 0.25


def _dist_body(write_en, z_ref, e_ref, idx_ref, en_ref, loss_ref):
    zb = z_ref[0]
    e = e_ref[...]
    en = e / jnp.maximum(
        jnp.sqrt(jnp.sum(e * e, axis=1, keepdims=True)), 1e-12)
    e2 = jnp.sum(en * en, axis=1)

    nz = jnp.sqrt(jnp.sum(zb * zb, axis=0, keepdims=True))
    zn = zb / jnp.maximum(nz, 1e-12)
    zr = zn.T
    z2 = jnp.sum(zr * zr, axis=1, keepdims=True)

    scores = jnp.dot(zr, en.T, preferred_element_type=jnp.float32)
    dist = z2 + e2[None, :] - 2.0 * scores

    m = jnp.min(dist, axis=1, keepdims=True)
    iota = jax.lax.broadcasted_iota(jnp.int32, dist.shape, 1)
    idx = jnp.min(jnp.where(dist == m, iota, jnp.int32(2**30)), axis=1)
    idx_ref[...] = idx
    loss_ref[0] = jnp.full((1, 128), jnp.sum(m), jnp.float32)

    if write_en:
        @pl.when(pl.program_id(0) == 0)
        def _():
            en_ref[...] = en


def _gather_body(en_hbm, idx_hbm, rows_hbm, idx_v, rows_v, sem):
    wid = lax.axis_index("s") * 2 + lax.axis_index("c")
    base = wid * 128
    pltpu.sync_copy(idx_hbm.at[pl.ds(base, 128)], idx_v)
    pltpu.async_copy(en_hbm.at[idx_v], rows_v, sem).wait()
    pltpu.sync_copy(rows_v, rows_hbm.at[pl.ds(base, 128)])


def _transpose_body(rows_ref, out_ref):
    out_ref[0] = rows_ref[0].T
    out_ref[1] = rows_ref[1].T


def _transpose_alias_body(prev_ref, rows_ref, out_ref):
    del prev_ref
    out_ref[0] = rows_ref[0].T
    out_ref[1] = rows_ref[1].T


def _dist_half(z3, embedding, b0, nb, write_en):
    d = z3.shape[1]
    p = z3.shape[2]
    k = embedding.shape[0]
    body = functools.partial(_dist_body, write_en)
    out_shape = [
        jax.ShapeDtypeStruct((nb * p,), jnp.int32),
        jax.ShapeDtypeStruct((k, d), jnp.float32),
        jax.ShapeDtypeStruct((nb, 1, 128), jnp.float32),
    ]
    res = pl.pallas_call(
        body,
        grid=(nb,),
        in_specs=[
            pl.BlockSpec((1, d, p), lambda i: (i + b0, 0, 0)),
            pl.BlockSpec((k, d), lambda i: (0, 0)),
        ],
        out_specs=[
            pl.BlockSpec((p,), lambda i: (i,)),
            pl.BlockSpec((k, d), lambda i: (0, 0)),
            pl.BlockSpec((1, 1, 128), lambda i: (i, 0, 0)),
        ],
        out_shape=out_shape,
        compiler_params=pltpu.CompilerParams(
            dimension_semantics=("arbitrary",)),
    )(z3, embedding)
    return res


def _gather_half(en_f32, idx_half, d):
    n = idx_half.shape[0]
    gather = functools.partial(
        pl.kernel,
        out_type=jax.ShapeDtypeStruct((n, d), jnp.float32),
        mesh=plsc.VectorSubcoreMesh(core_axis_name="c", subcore_axis_name="s"),
        scratch_types=[
            pltpu.VMEM((n // 32,), jnp.int32),
            pltpu.VMEM((n // 32, d), jnp.float32),
            pltpu.SemaphoreType.DMA,
        ],
    )(_gather_body)
    return gather(en_f32, idx_half)


@jax.jit
def kernel(z, embedding):
    b, d, h, w = z.shape
    k = embedding.shape[0]
    p = h * w
    hb = b // 2
    z3 = z.reshape(b, d, p)

    idx_a, en_f32, loss_a = _dist_half(z3, embedding, 0, hb, True)
    idx_b, _, loss_b = _dist_half(z3, embedding, hb, hb, False)

    rows_a = _gather_half(en_f32, idx_a, d)
    rows_b = _gather_half(en_f32, idx_b, d)

    out_a = pl.pallas_call(
        _transpose_body,
        grid=(hb // 2,),
        in_specs=[pl.BlockSpec((2, p, d), lambda i: (i, 0, 0))],
        out_specs=pl.BlockSpec((2, d, p), lambda i: (i, 0, 0)),
        out_shape=jax.ShapeDtypeStruct((b, d, p), jnp.float32),
        compiler_params=pltpu.CompilerParams(
            dimension_semantics=("arbitrary",)),
    )(rows_a.reshape(hb, p, d))

    out = pl.pallas_call(
        _transpose_alias_body,
        grid=(hb // 2,),
        in_specs=[
            pl.BlockSpec(memory_space=pl.ANY),
            pl.BlockSpec((2, p, d), lambda i: (i, 0, 0)),
        ],
        out_specs=pl.BlockSpec((2, d, p), lambda i: (i + 2, 0, 0)),
        out_shape=jax.ShapeDtypeStruct((b, d, p), jnp.float32),
        input_output_aliases={0: 0},
        compiler_params=pltpu.CompilerParams(
            dimension_semantics=("arbitrary",)),
    )(out_a, rows_b.reshape(hb, p, d))

    idx = jnp.concatenate([idx_a, idx_b])
    total = jnp.sum(loss_a[:, 0, 0]) + jnp.sum(loss_b[:, 0, 0])
    mean_sq = total / jnp.float32(b * p * d)
    loss = mean_sq + _BETA * mean_sq
    return (out.reshape(b, d, h, w), idx, loss)


# speedup vs baseline: 1.1565x; 1.0166x over previous
import functools

import jax
import jax.numpy as jnp
from jax import lax
from jax.experimental import pallas as pl
from jax.experimental.pallas import tpu as pltpu
from jax.experimental.pallas import tpu_sc as plsc

_BETA = 0.25


def _dist_body(write_en, z_ref, e_ref, idx_ref, en_ref, loss_ref):
    e = e_ref[...]
    en = e / jnp.maximum(
        jnp.sqrt(jnp.sum(e * e, axis=1, keepdims=True)), 1e-12)
    e2 = jnp.sum(en * en, axis=1)

    for sub in range(2):
        zb = z_ref[sub]
        nz = jnp.sqrt(jnp.sum(zb * zb, axis=0, keepdims=True))
        zn = zb / jnp.maximum(nz, 1e-12)
        zr = zn.T
        z2 = jnp.sum(zr * zr, axis=1, keepdims=True)

        scores = jnp.dot(zr, en.T,
                         preferred_element_type=jnp.float32)
        dist = z2 + e2[None, :] - 2.0 * scores

        m = jnp.min(dist, axis=1, keepdims=True)
        iota = jax.lax.broadcasted_iota(jnp.int32, dist.shape, 1)
        idx = jnp.min(jnp.where(dist == m, iota, jnp.int32(2**30)),
                      axis=1)
        p = idx.shape[0]
        idx_ref[pl.ds(sub * p, p)] = idx
        loss_ref[sub] = jnp.full((1, 128), jnp.sum(m), jnp.float32)

    if write_en:
        @pl.when(pl.program_id(0) == 0)
        def _():
            en_ref[...] = en


def _gather_body(en_hbm, idx_hbm, rows_hbm, idx_v, rows_v, sem):
    wid = lax.axis_index("s") * 2 + lax.axis_index("c")
    base = wid * 128
    pltpu.sync_copy(idx_hbm.at[pl.ds(base, 128)], idx_v)
    pltpu.async_copy(en_hbm.at[idx_v], rows_v, sem).wait()
    pltpu.sync_copy(rows_v, rows_hbm.at[pl.ds(base, 128)])


def _transpose_body(rows_ref, out_ref):
    out_ref[0] = rows_ref[0].T
    out_ref[1] = rows_ref[1].T


def _transpose_alias_body(prev_ref, rows_ref, out_ref):
    del prev_ref
    out_ref[0] = rows_ref[0].T
    out_ref[1] = rows_ref[1].T


def _dist_half(z3, embedding, b0, nb, write_en):
    d = z3.shape[1]
    p = z3.shape[2]
    k = embedding.shape[0]
    body = functools.partial(_dist_body, write_en)
    out_shape = [
        jax.ShapeDtypeStruct((nb * p,), jnp.int32),
        jax.ShapeDtypeStruct((k, d), jnp.float32),
        jax.ShapeDtypeStruct((nb, 1, 128), jnp.float32),
    ]
    res = pl.pallas_call(
        body,
        grid=(nb // 2,),
        in_specs=[
            pl.BlockSpec((2, d, p), lambda i: (i + b0 // 2, 0, 0)),
            pl.BlockSpec((k, d), lambda i: (0, 0)),
        ],
        out_specs=[
            pl.BlockSpec((2 * p,), lambda i: (i,)),
            pl.BlockSpec((k, d), lambda i: (0, 0)),
            pl.BlockSpec((2, 1, 128), lambda i: (i, 0, 0)),
        ],
        out_shape=out_shape,
        compiler_params=pltpu.CompilerParams(
            dimension_semantics=("arbitrary",)),
    )(z3, embedding)
    return res


def _gather_half(en_f32, idx_half, d):
    n = idx_half.shape[0]
    gather = functools.partial(
        pl.kernel,
        out_type=jax.ShapeDtypeStruct((n, d), jnp.float32),
        mesh=plsc.VectorSubcoreMesh(core_axis_name="c", subcore_axis_name="s"),
        scratch_types=[
            pltpu.VMEM((n // 32,), jnp.int32),
            pltpu.VMEM((n // 32, d), jnp.float32),
            pltpu.SemaphoreType.DMA,
        ],
    )(_gather_body)
    return gather(en_f32, idx_half)


@jax.jit
def kernel(z, embedding):
    b, d, h, w = z.shape
    k = embedding.shape[0]
    p = h * w
    hb = b // 2
    z3 = z.reshape(b, d, p)

    idx_a, en_f32, loss_a = _dist_half(z3, embedding, 0, hb, True)
    idx_b, _, loss_b = _dist_half(z3, embedding, hb, hb, False)

    rows_a = _gather_half(en_f32, idx_a, d)
    rows_b = _gather_half(en_f32, idx_b, d)

    out_a = pl.pallas_call(
        _transpose_body,
        grid=(hb // 2,),
        in_specs=[pl.BlockSpec((2, p, d), lambda i: (i, 0, 0))],
        out_specs=pl.BlockSpec((2, d, p), lambda i: (i, 0, 0)),
        out_shape=jax.ShapeDtypeStruct((b, d, p), jnp.float32),
        compiler_params=pltpu.CompilerParams(
            dimension_semantics=("arbitrary",)),
    )(rows_a.reshape(hb, p, d))

    out = pl.pallas_call(
        _transpose_alias_body,
        grid=(hb // 2,),
        in_specs=[
            pl.BlockSpec(memory_space=pl.ANY),
            pl.BlockSpec((2, p, d), lambda i: (i, 0, 0)),
        ],
        out_specs=pl.BlockSpec((2, d, p), lambda i: (i + 2, 0, 0)),
        out_shape=jax.ShapeDtypeStruct((b, d, p), jnp.float32),
        input_output_aliases={0: 0},
        compiler_params=pltpu.CompilerParams(
            dimension_semantics=("arbitrary",)),
    )(out_a, rows_b.reshape(hb, p, d))

    idx = jnp.concatenate([idx_a, idx_b])
    total = jnp.sum(loss_a[:, 0, 0]) + jnp.sum(loss_b[:, 0, 0])
    mean_sq = total / jnp.float32(b * p * d)
    loss = mean_sq + _BETA * mean_sq
    return (out.reshape(b, d, h, w), idx, loss)


# speedup vs baseline: 1.7370x; 1.5020x over previous
import functools

import jax
import jax.numpy as jnp
from jax import lax
from jax.experimental import pallas as pl
from jax.experimental.pallas import tpu as pltpu
from jax.experimental.pallas import tpu_sc as plsc

_BETA = 0.25


def _dist_body(z_ref, e_ref, idx_ref, en_ref, loss_ref):
    e = e_ref[...]
    en = e / jnp.maximum(
        jnp.sqrt(jnp.sum(e * e, axis=1, keepdims=True)), 1e-12)
    e2 = jnp.sum(en * en, axis=1)

    for sub in range(2):
        zr = z_ref[sub]
        nz = jnp.sqrt(jnp.sum(zr * zr, axis=1, keepdims=True))
        zn = zr / jnp.maximum(nz, 1e-12)
        z2 = jnp.sum(zn * zn, axis=1, keepdims=True)

        scores = jnp.dot(zn, en.T,
                         preferred_element_type=jnp.float32)
        dist = z2 + e2[None, :] - 2.0 * scores

        m = jnp.min(dist, axis=1, keepdims=True)
        iota = jax.lax.broadcasted_iota(jnp.int32, dist.shape, 1)
        idx = jnp.min(jnp.where(dist == m, iota, jnp.int32(2**30)),
                      axis=1)
        p = idx.shape[0]
        idx_ref[pl.ds(sub * p, p)] = idx
        loss_ref[sub] = jnp.full((1, 128), jnp.sum(m), jnp.float32)

    @pl.when(pl.program_id(0) == 0)
    def _():
        en_ref[...] = en


def _gather_body(en_hbm, idx_hbm, rows_hbm, idx_v, rows_v, sem):
    wid = lax.axis_index("s") * 2 + lax.axis_index("c")
    base = wid * 256
    pltpu.sync_copy(idx_hbm.at[pl.ds(base, 256)], idx_v)
    pltpu.async_copy(en_hbm.at[idx_v], rows_v, sem).wait()
    pltpu.sync_copy(rows_v, rows_hbm.at[pl.ds(base, 256)])


@jax.jit
def kernel(z, embedding):
    b, d, h, w = z.shape
    k = embedding.shape[0]
    p = h * w
    zrows = jnp.transpose(z.reshape(b, d, p), (0, 2, 1))

    idx, en, losssum = pl.pallas_call(
        _dist_body,
        grid=(b // 2,),
        in_specs=[
            pl.BlockSpec((2, p, d), lambda i: (i, 0, 0)),
            pl.BlockSpec((k, d), lambda i: (0, 0)),
        ],
        out_specs=[
            pl.BlockSpec((2 * p,), lambda i: (i,)),
            pl.BlockSpec((k, d), lambda i: (0, 0)),
            pl.BlockSpec((2, 1, 128), lambda i: (i, 0, 0)),
        ],
        out_shape=[
            jax.ShapeDtypeStruct((b * p,), jnp.int32),
            jax.ShapeDtypeStruct((k, d), jnp.float32),
            jax.ShapeDtypeStruct((b, 1, 128), jnp.float32),
        ],
        compiler_params=pltpu.CompilerParams(
            dimension_semantics=("arbitrary",)),
    )(zrows, embedding)

    gather = functools.partial(
        pl.kernel,
        out_type=jax.ShapeDtypeStruct((b * p, d), jnp.float32),
        mesh=plsc.VectorSubcoreMesh(core_axis_name="c", subcore_axis_name="s"),
        scratch_types=[
            pltpu.VMEM((256,), jnp.int32),
            pltpu.VMEM((256, d), jnp.float32),
            pltpu.SemaphoreType.DMA,
        ],
    )(_gather_body)
    rows = gather(en, idx)

    out = jnp.transpose(rows.reshape(b, p, d), (0, 2, 1)).reshape(b, d, h, w)

    mean_sq = jnp.sum(losssum[:, 0, 0]) / jnp.float32(b * p * d)
    loss = mean_sq + _BETA * mean_sq
    return (out, idx, loss)
